# baseline (device time: 80403 ns/iter reference)
import jax
import jax.numpy as jnp
from jax import lax
from jax.experimental import pallas as pl
from jax.experimental.pallas import tpu as pltpu

N_DEV = 4
SQ = 512
D = 1024
H = 8
DH = 128
SCALE = 0.08838834764831843


def kernel(x, Wq, Wo, Wk, Wv):
    def body(
        x_ref, wq_ref, wo_ref, wk_ref, wv_ref, out_ref,
        xg_ref, wqb_ref, wkb_ref, wvb_ref, wob_ref, o_scr,
        p_send_ref, p_recv_ref,
        x_send_sems, x_recv_sems, p_send_sems, p_recv_sems,
    ):
        my = lax.axis_index("i")

        barrier = pltpu.get_barrier_semaphore()
        for s in range(1, N_DEV):
            pl.semaphore_signal(
                barrier, inc=1,
                device_id=((my + s) % N_DEV,),
                device_id_type=pl.DeviceIdType.MESH,
            )
        pl.semaphore_wait(barrier, N_DEV - 1)

        xg_ref[my] = x_ref[0].astype(jnp.bfloat16)
        x_rdmas = []
        for s in range(1, N_DEV):
            t = (my + s) % N_DEV
            rd = pltpu.make_async_remote_copy(
                src_ref=xg_ref.at[my],
                dst_ref=xg_ref.at[my],
                send_sem=x_send_sems.at[s - 1],
                recv_sem=x_recv_sems.at[my],
                device_id=(t,),
                device_id_type=pl.DeviceIdType.MESH,
            )
            rd.start()
            x_rdmas.append(rd)

        wqb_ref[...] = wq_ref[...].astype(jnp.bfloat16)
        wkb_ref[...] = wk_ref[...].astype(jnp.bfloat16)
        wvb_ref[...] = wv_ref[...].astype(jnp.bfloat16)
        wob_ref[...] = wo_ref[...].astype(jnp.bfloat16)

        def compute_batch(slot):
            xb = xg_ref[slot]
            q = jnp.dot(xb, wqb_ref[...], preferred_element_type=jnp.float32)
            k = jnp.dot(xb, wkb_ref[...], preferred_element_type=jnp.float32)
            v = jnp.dot(xb, wvb_ref[...], preferred_element_type=jnp.float32)
            q = q.astype(jnp.bfloat16)
            k = k.astype(jnp.bfloat16)
            v = v.astype(jnp.bfloat16)
            for h in range(H):
                sl = slice(h * DH, (h + 1) * DH)
                sc = lax.dot_general(
                    q[:, sl], k[:, sl],
                    dimension_numbers=(((1,), (1,)), ((), ())),
                    preferred_element_type=jnp.float32,
                ) * SCALE
                m = jnp.max(sc, axis=1, keepdims=True)
                p = jnp.exp(sc - m)
                l = jnp.sum(p, axis=1, keepdims=True)
                o = jnp.dot(
                    p.astype(jnp.bfloat16), v[:, sl],
                    preferred_element_type=jnp.float32,
                ) / l
                o_scr[:, sl] = o.astype(jnp.bfloat16)
            return jnp.dot(
                o_scr[...], wob_ref[...], preferred_element_type=jnp.float32
            )

        out_ref[0] = compute_batch(my)

        p_rdmas = []
        for s in range(1, N_DEV):
            t = (my + s) % N_DEV
            recv_x = pltpu.make_async_remote_copy(
                src_ref=xg_ref.at[t],
                dst_ref=xg_ref.at[t],
                send_sem=x_send_sems.at[0],
                recv_sem=x_recv_sems.at[t],
                device_id=(t,),
                device_id_type=pl.DeviceIdType.MESH,
            )
            recv_x.wait_recv()
            p_send_ref[s - 1] = compute_batch(t).astype(jnp.bfloat16)
            rd = pltpu.make_async_remote_copy(
                src_ref=p_send_ref.at[s - 1],
                dst_ref=p_recv_ref.at[my],
                send_sem=p_send_sems.at[s - 1],
                recv_sem=p_recv_sems.at[my],
                device_id=(t,),
                device_id_type=pl.DeviceIdType.MESH,
            )
            rd.start()
            p_rdmas.append(rd)

        for s in range(1, N_DEV):
            t = (my + s) % N_DEV
            recv_p = pltpu.make_async_remote_copy(
                src_ref=p_recv_ref.at[t],
                dst_ref=p_recv_ref.at[t],
                send_sem=p_send_sems.at[0],
                recv_sem=p_recv_sems.at[t],
                device_id=(t,),
                device_id_type=pl.DeviceIdType.MESH,
            )
            recv_p.wait_recv()
            out_ref[0] = out_ref[0] + p_recv_ref[t].astype(jnp.float32)

        for rd in x_rdmas + p_rdmas:
            rd.wait_send()

    return pl.pallas_call(
        body,
        out_shape=jax.ShapeDtypeStruct((1, SQ, D), jnp.float32),
        in_specs=[pl.BlockSpec(memory_space=pltpu.VMEM)] * 5,
        out_specs=pl.BlockSpec(memory_space=pltpu.VMEM),
        scratch_shapes=[
            pltpu.VMEM((N_DEV, SQ, D), jnp.bfloat16),
            pltpu.VMEM((D, D), jnp.bfloat16),
            pltpu.VMEM((D, D), jnp.bfloat16),
            pltpu.VMEM((D, D), jnp.bfloat16),
            pltpu.VMEM((D, D), jnp.bfloat16),
            pltpu.VMEM((SQ, D), jnp.bfloat16),
            pltpu.VMEM((N_DEV - 1, SQ, D), jnp.bfloat16),
            pltpu.VMEM((N_DEV, SQ, D), jnp.bfloat16),
            pltpu.SemaphoreType.DMA((N_DEV - 1,)),
            pltpu.SemaphoreType.DMA((N_DEV,)),
            pltpu.SemaphoreType.DMA((N_DEV - 1,)),
            pltpu.SemaphoreType.DMA((N_DEV,)),
        ],
        compiler_params=pltpu.CompilerParams(collective_id=0),
    )(x, Wq, Wo, Wk, Wv)


# device time: 73828 ns/iter; 1.0891x vs baseline; 1.0891x over previous
import jax
import jax.numpy as jnp
from jax import lax
from jax.experimental import pallas as pl
from jax.experimental.pallas import tpu as pltpu

N_DEV = 4
SQ = 512
D = 1024
H = 8
DH = 128
SCALE = 0.08838834764831843


def kernel(x, Wq, Wo, Wk, Wv):
    def body(
        x_ref, wq_ref, wo_ref, wk_ref, wv_ref, out_ref,
        xg_ref, wqb_ref, wkb_ref, wvb_ref, wob_ref, o_scr,
        p_send_ref, p_recv_ref,
        x_send_sems, x_recv_sems, p_send_sems, p_recv_sems,
    ):
        my = lax.axis_index("i")

        barrier = pltpu.get_barrier_semaphore()
        for s in range(1, N_DEV):
            pl.semaphore_signal(
                barrier, inc=1,
                device_id=((my + s) % N_DEV,),
                device_id_type=pl.DeviceIdType.MESH,
            )
        pl.semaphore_wait(barrier, N_DEV - 1)

        xg_ref[my] = x_ref[0].astype(jnp.bfloat16)
        x_rdmas = []
        for s in range(N_DEV - 1, 0, -1):
            t = (my + s) % N_DEV
            rd = pltpu.make_async_remote_copy(
                src_ref=xg_ref.at[my],
                dst_ref=xg_ref.at[my],
                send_sem=x_send_sems.at[s - 1],
                recv_sem=x_recv_sems.at[my],
                device_id=(t,),
                device_id_type=pl.DeviceIdType.MESH,
            )
            rd.start()
            x_rdmas.append(rd)

        wqb_ref[...] = wq_ref[...].astype(jnp.bfloat16)
        wkb_ref[...] = wk_ref[...].astype(jnp.bfloat16)
        wvb_ref[...] = wv_ref[...].astype(jnp.bfloat16)
        wob_ref[...] = wo_ref[...].astype(jnp.bfloat16)

        def compute_batch(slot):
            xb = xg_ref[slot]
            q = jnp.dot(xb, wqb_ref[...], preferred_element_type=jnp.float32)
            k = jnp.dot(xb, wkb_ref[...], preferred_element_type=jnp.float32)
            v = jnp.dot(xb, wvb_ref[...], preferred_element_type=jnp.float32)
            q = q.astype(jnp.bfloat16)
            k = k.astype(jnp.bfloat16)
            v = v.astype(jnp.bfloat16)
            for h in range(H):
                sl = slice(h * DH, (h + 1) * DH)
                sc = lax.dot_general(
                    q[:, sl], k[:, sl],
                    dimension_numbers=(((1,), (1,)), ((), ())),
                    preferred_element_type=jnp.float32,
                ) * SCALE
                m = jnp.max(sc, axis=1, keepdims=True)
                p = jnp.exp(sc - m)
                l = jnp.sum(p, axis=1, keepdims=True)
                o = jnp.dot(
                    p.astype(jnp.bfloat16), v[:, sl],
                    preferred_element_type=jnp.float32,
                ) / l
                o_scr[:, sl] = o.astype(jnp.bfloat16)
            return jnp.dot(
                o_scr[...], wob_ref[...], preferred_element_type=jnp.float32
            )

        out_ref[0] = compute_batch(my)

        p_rdmas = []
        for s in range(1, N_DEV):
            t = (my + s) % N_DEV
            recv_x = pltpu.make_async_remote_copy(
                src_ref=xg_ref.at[t],
                dst_ref=xg_ref.at[t],
                send_sem=x_send_sems.at[0],
                recv_sem=x_recv_sems.at[t],
                device_id=(t,),
                device_id_type=pl.DeviceIdType.MESH,
            )
            recv_x.wait_recv()
            p_send_ref[s - 1] = compute_batch(t).astype(jnp.bfloat16)
            rd = pltpu.make_async_remote_copy(
                src_ref=p_send_ref.at[s - 1],
                dst_ref=p_recv_ref.at[my],
                send_sem=p_send_sems.at[s - 1],
                recv_sem=p_recv_sems.at[my],
                device_id=(t,),
                device_id_type=pl.DeviceIdType.MESH,
            )
            rd.start()
            p_rdmas.append(rd)

        for s in range(N_DEV - 1, 0, -1):
            t = (my + s) % N_DEV
            recv_p = pltpu.make_async_remote_copy(
                src_ref=p_recv_ref.at[t],
                dst_ref=p_recv_ref.at[t],
                send_sem=p_send_sems.at[0],
                recv_sem=p_recv_sems.at[t],
                device_id=(t,),
                device_id_type=pl.DeviceIdType.MESH,
            )
            recv_p.wait_recv()
            out_ref[0] = out_ref[0] + p_recv_ref[t].astype(jnp.float32)

        for rd in x_rdmas + p_rdmas:
            rd.wait_send()

    return pl.pallas_call(
        body,
        out_shape=jax.ShapeDtypeStruct((1, SQ, D), jnp.float32),
        in_specs=[pl.BlockSpec(memory_space=pltpu.VMEM)] * 5,
        out_specs=pl.BlockSpec(memory_space=pltpu.VMEM),
        scratch_shapes=[
            pltpu.VMEM((N_DEV, SQ, D), jnp.bfloat16),
            pltpu.VMEM((D, D), jnp.bfloat16),
            pltpu.VMEM((D, D), jnp.bfloat16),
            pltpu.VMEM((D, D), jnp.bfloat16),
            pltpu.VMEM((D, D), jnp.bfloat16),
            pltpu.VMEM((SQ, D), jnp.bfloat16),
            pltpu.VMEM((N_DEV - 1, SQ, D), jnp.bfloat16),
            pltpu.VMEM((N_DEV, SQ, D), jnp.bfloat16),
            pltpu.SemaphoreType.DMA((N_DEV - 1,)),
            pltpu.SemaphoreType.DMA((N_DEV,)),
            pltpu.SemaphoreType.DMA((N_DEV - 1,)),
            pltpu.SemaphoreType.DMA((N_DEV,)),
        ],
        compiler_params=pltpu.CompilerParams(collective_id=0),
    )(x, Wq, Wo, Wk, Wv)


# device time: 73438 ns/iter; 1.0948x vs baseline; 1.0053x over previous
import jax
import jax.numpy as jnp
from jax import lax
from jax.experimental import pallas as pl
from jax.experimental.pallas import tpu as pltpu

N_DEV = 4
SQ = 512
D = 1024
H = 8
DH = 128
SCALE = 0.08838834764831843


def kernel(x, Wq, Wo, Wk, Wv):
    def body(
        x_ref, wq_ref, wo_ref, wk_ref, wv_ref, out_ref,
        xg_ref, wqb_ref, wkb_ref, wvb_ref, wob_ref, o_scr,
        p_send_ref, p_recv_ref,
        x_send_sems, x_recv_sems, p_send_sems, p_recv_sems,
    ):
        my = lax.axis_index("i")

        barrier = pltpu.get_barrier_semaphore()
        for s in range(1, N_DEV):
            pl.semaphore_signal(
                barrier, inc=1,
                device_id=((my + s) % N_DEV,),
                device_id_type=pl.DeviceIdType.MESH,
            )
        pl.semaphore_wait(barrier, N_DEV - 1)

        xg_ref[my] = x_ref[0].astype(jnp.bfloat16)
        x_rdmas = []
        for s in range(N_DEV - 1, 0, -1):
            t = (my + s) % N_DEV
            rd = pltpu.make_async_remote_copy(
                src_ref=xg_ref.at[my],
                dst_ref=xg_ref.at[my],
                send_sem=x_send_sems.at[s - 1],
                recv_sem=x_recv_sems.at[my],
                device_id=(t,),
                device_id_type=pl.DeviceIdType.MESH,
            )
            rd.start()
            x_rdmas.append(rd)

        wqb_ref[...] = (wq_ref[...] * SCALE).astype(jnp.bfloat16)
        wkb_ref[...] = wk_ref[...].astype(jnp.bfloat16)
        wvb_ref[...] = wv_ref[...].astype(jnp.bfloat16)
        wob_ref[...] = wo_ref[...].astype(jnp.bfloat16)

        def compute_batch(slot):
            xb = xg_ref[slot]
            q = jnp.dot(
                xb, wqb_ref[...], preferred_element_type=jnp.float32
            ).astype(jnp.bfloat16)
            k = jnp.dot(
                xb, wkb_ref[...], preferred_element_type=jnp.float32
            ).astype(jnp.bfloat16)
            v = jnp.dot(
                xb, wvb_ref[...], preferred_element_type=jnp.float32
            ).astype(jnp.bfloat16)
            for h in range(H):
                sl = slice(h * DH, (h + 1) * DH)
                sc = lax.dot_general(
                    q[:, sl], k[:, sl],
                    dimension_numbers=(((1,), (1,)), ((), ())),
                    preferred_element_type=jnp.float32,
                )
                m = jnp.max(sc, axis=1, keepdims=True)
                p = jnp.exp(sc - m)
                l = jnp.sum(p, axis=1, keepdims=True)
                o = jnp.dot(
                    p.astype(jnp.bfloat16), v[:, sl],
                    preferred_element_type=jnp.float32,
                ) / l
                o_scr[:, sl] = o.astype(jnp.bfloat16)
            return jnp.dot(
                o_scr[...], wob_ref[...], preferred_element_type=jnp.float32
            )

        out_ref[0] = compute_batch(my)

        p_rdmas = []
        for s in range(1, N_DEV):
            t = (my + s) % N_DEV
            recv_x = pltpu.make_async_remote_copy(
                src_ref=xg_ref.at[t],
                dst_ref=xg_ref.at[t],
                send_sem=x_send_sems.at[0],
                recv_sem=x_recv_sems.at[t],
                device_id=(t,),
                device_id_type=pl.DeviceIdType.MESH,
            )
            recv_x.wait_recv()
            p_send_ref[s - 1] = compute_batch(t).astype(jnp.bfloat16)
            rd = pltpu.make_async_remote_copy(
                src_ref=p_send_ref.at[s - 1],
                dst_ref=p_recv_ref.at[my],
                send_sem=p_send_sems.at[s - 1],
                recv_sem=p_recv_sems.at[my],
                device_id=(t,),
                device_id_type=pl.DeviceIdType.MESH,
            )
            rd.start()
            p_rdmas.append(rd)

        for s in range(N_DEV - 1, 0, -1):
            t = (my + s) % N_DEV
            recv_p = pltpu.make_async_remote_copy(
                src_ref=p_recv_ref.at[t],
                dst_ref=p_recv_ref.at[t],
                send_sem=p_send_sems.at[0],
                recv_sem=p_recv_sems.at[t],
                device_id=(t,),
                device_id_type=pl.DeviceIdType.MESH,
            )
            recv_p.wait_recv()
            out_ref[0] = out_ref[0] + p_recv_ref[t].astype(jnp.float32)

        for rd in x_rdmas + p_rdmas:
            rd.wait_send()

    return pl.pallas_call(
        body,
        out_shape=jax.ShapeDtypeStruct((1, SQ, D), jnp.float32),
        in_specs=[pl.BlockSpec(memory_space=pltpu.VMEM)] * 5,
        out_specs=pl.BlockSpec(memory_space=pltpu.VMEM),
        scratch_shapes=[
            pltpu.VMEM((N_DEV, SQ, D), jnp.bfloat16),
            pltpu.VMEM((D, D), jnp.bfloat16),
            pltpu.VMEM((D, D), jnp.bfloat16),
            pltpu.VMEM((D, D), jnp.bfloat16),
            pltpu.VMEM((D, D), jnp.bfloat16),
            pltpu.VMEM((SQ, D), jnp.bfloat16),
            pltpu.VMEM((N_DEV - 1, SQ, D), jnp.bfloat16),
            pltpu.VMEM((N_DEV, SQ, D), jnp.bfloat16),
            pltpu.SemaphoreType.DMA((N_DEV - 1,)),
            pltpu.SemaphoreType.DMA((N_DEV,)),
            pltpu.SemaphoreType.DMA((N_DEV - 1,)),
            pltpu.SemaphoreType.DMA((N_DEV,)),
        ],
        compiler_params=pltpu.CompilerParams(collective_id=0),
    )(x, Wq, Wo, Wk, Wv)
